# baseline (speedup 1.0000x reference)
import functools

import jax
import jax.numpy as jnp
from jax.experimental import pallas as pl
from jax.experimental.pallas import tpu as pltpu


def _fold_kernel(ws_ref, wn_ref, wc_ref, bs_ref, bn_ref, bc_ref,
                 at_ref, bt_ref, c_ref):
    out = ws_ref.shape[0]
    wc1 = wc_ref[:, :out]
    wc2 = wc_ref[:, out:]
    at_ref[...] = jax.lax.dot_general(
        ws_ref[...], wc1, (((0,), (1,)), ((), ())),
        preferred_element_type=jnp.float32)
    bt_ref[...] = jax.lax.dot_general(
        wn_ref[...], wc2, (((0,), (1,)), ((), ())),
        preferred_element_type=jnp.float32)
    c_ref[...] = (bc_ref[...]
                  + jax.lax.dot_general(bs_ref[...], wc1,
                                        (((1,), (1,)), ((), ())),
                                        preferred_element_type=jnp.float32)
                  + jax.lax.dot_general(bn_ref[...], wc2,
                                        (((1,), (1,)), ((), ())),
                                        preferred_element_type=jnp.float32))


def _main_kernel(adj_ref, x_ref, at_ref, bt_ref, c_ref, out_ref):
    m = adj_ref.shape[0]
    i = pl.program_id(0)
    mask = (adj_ref[...] > 0).astype(jnp.bfloat16)
    deg = jnp.sum(mask.astype(jnp.float32), axis=1, keepdims=True)
    scale = 1.0 / jnp.maximum(deg, 1.0)
    agg = jnp.dot(mask, x_ref[...].astype(jnp.bfloat16),
                  preferred_element_type=jnp.float32)
    agg = agg * scale
    x_tile = x_ref[pl.ds(i * m, m), :]
    y = jnp.dot(x_tile, at_ref[...], preferred_element_type=jnp.float32)
    y = y + jnp.dot(agg, bt_ref[...], preferred_element_type=jnp.float32)
    y = y + c_ref[...]
    out_ref[...] = jnp.maximum(y, 0.0)


@functools.partial(jax.jit, static_argnames=())
def kernel(x, adj, W_self, b_self, W_neigh, b_neigh, W_comb, b_comb):
    n, d = x.shape
    out = W_self.shape[0]

    at, bt, c = pl.pallas_call(
        _fold_kernel,
        out_shape=[
            jax.ShapeDtypeStruct((d, out), jnp.float32),
            jax.ShapeDtypeStruct((d, out), jnp.float32),
            jax.ShapeDtypeStruct((1, out), jnp.float32),
        ],
    )(W_self, W_neigh, W_comb,
      b_self.reshape(1, out), b_neigh.reshape(1, out), b_comb.reshape(1, out))

    m = 256
    grid = (n // m,)
    y = pl.pallas_call(
        _main_kernel,
        grid=grid,
        in_specs=[
            pl.BlockSpec((m, n), lambda i: (i, 0)),
            pl.BlockSpec((n, d), lambda i: (0, 0)),
            pl.BlockSpec((d, out), lambda i: (0, 0)),
            pl.BlockSpec((d, out), lambda i: (0, 0)),
            pl.BlockSpec((1, out), lambda i: (0, 0)),
        ],
        out_specs=pl.BlockSpec((m, out), lambda i: (i, 0)),
        out_shape=jax.ShapeDtypeStruct((n, out), jnp.float32),
        compiler_params=pltpu.CompilerParams(
            dimension_semantics=("parallel",)),
    )(adj, x, at, bt, c)
    return y


# speedup vs baseline: 1.1271x; 1.1271x over previous
import functools

import jax
import jax.numpy as jnp
from jax.experimental import pallas as pl
from jax.experimental.pallas import tpu as pltpu


def _fold_kernel(ws_ref, wn_ref, wc_ref, bs_ref, bn_ref, bc_ref,
                 at_ref, bt_ref, c_ref):
    out = ws_ref.shape[0]
    wc1 = wc_ref[:, :out]
    wc2 = wc_ref[:, out:]
    at_ref[...] = jax.lax.dot_general(
        ws_ref[...], wc1, (((0,), (1,)), ((), ())),
        preferred_element_type=jnp.float32)
    bt_ref[...] = jax.lax.dot_general(
        wn_ref[...], wc2, (((0,), (1,)), ((), ())),
        preferred_element_type=jnp.float32)
    c_ref[...] = (bc_ref[...]
                  + jax.lax.dot_general(bs_ref[...], wc1,
                                        (((1,), (1,)), ((), ())),
                                        preferred_element_type=jnp.float32)
                  + jax.lax.dot_general(bn_ref[...], wc2,
                                        (((1,), (1,)), ((), ())),
                                        preferred_element_type=jnp.float32))


def _main_kernel(adj_ref, x_ref, at_ref, bt_ref, c_ref, out_ref):
    m = adj_ref.shape[0]
    i = pl.program_id(0)
    mask = (adj_ref[...] > 0).astype(jnp.bfloat16)
    deg = jnp.sum(mask.astype(jnp.float32), axis=1, keepdims=True)
    scale = 1.0 / jnp.maximum(deg, 1.0)
    agg = jnp.dot(mask, x_ref[...].astype(jnp.bfloat16),
                  preferred_element_type=jnp.float32)
    agg = agg * scale
    x_tile = x_ref[pl.ds(i * m, m), :]
    y = jnp.dot(x_tile, at_ref[...], preferred_element_type=jnp.float32)
    y = y + jnp.dot(agg, bt_ref[...], preferred_element_type=jnp.float32)
    y = y + c_ref[...]
    out_ref[...] = jnp.maximum(y, 0.0)


@functools.partial(jax.jit, static_argnames=())
def kernel(x, adj, W_self, b_self, W_neigh, b_neigh, W_comb, b_comb):
    n, d = x.shape
    out = W_self.shape[0]

    at, bt, c = pl.pallas_call(
        _fold_kernel,
        out_shape=[
            jax.ShapeDtypeStruct((d, out), jnp.float32),
            jax.ShapeDtypeStruct((d, out), jnp.float32),
            jax.ShapeDtypeStruct((1, out), jnp.float32),
        ],
    )(W_self, W_neigh, W_comb,
      b_self.reshape(1, out), b_neigh.reshape(1, out), b_comb.reshape(1, out))

    m = 512
    grid = (n // m,)
    y = pl.pallas_call(
        _main_kernel,
        grid=grid,
        in_specs=[
            pl.BlockSpec((m, n), lambda i: (i, 0)),
            pl.BlockSpec((n, d), lambda i: (0, 0)),
            pl.BlockSpec((d, out), lambda i: (0, 0)),
            pl.BlockSpec((d, out), lambda i: (0, 0)),
            pl.BlockSpec((1, out), lambda i: (0, 0)),
        ],
        out_specs=pl.BlockSpec((m, out), lambda i: (i, 0)),
        out_shape=jax.ShapeDtypeStruct((n, out), jnp.float32),
        compiler_params=pltpu.CompilerParams(
            dimension_semantics=("parallel",)),
    )(adj, x, at, bt, c)
    return y
